# baseline (speedup 1.0000x reference)
import functools
import math

import jax
import jax.numpy as jnp
from jax import lax
from jax.experimental import pallas as pl
from jax.experimental.pallas import tpu as pltpu
from jax.experimental.pallas import tpu_sc as plsc

N = 20000
NUM_CLASSES = 81
IMG_W = 1024.0
IMG_H = 1024.0
SCORE_THRESH = 0.05
NMS_THRESH = 0.5
DETECTIONS_PER_IMG = 100
PRE_NMS_TOPK = 300
BBOX_XFORM_CLIP = math.log(1000.0 / 16.0)
WX, WY, WW, WH = 10.0, 10.0, 5.0, 5.0
TO_REMOVE = 1.0
C = NUM_CLASSES - 1

N_PAD = 20480
ROWS_PER_BLK = 2048
SORT_W = 1024
REG_B = 304
PAD_IDX = 1 << 29
NEG = -3.0e38


def _softmax_t_body(logits_ref, out_ref):
    i = pl.program_id(0)
    x = logits_ref[...]
    m = jnp.max(x, axis=-1, keepdims=True)
    e = jnp.exp(x - m)
    p = e / jnp.sum(e, axis=-1, keepdims=True)
    row = i * ROWS_PER_BLK + lax.broadcasted_iota(jnp.int32, x.shape, 0)
    p = jnp.where(row < N, p, -1.0)
    out_ref[...] = p.T


def _softmax_t(class_logits):
    return pl.pallas_call(
        _softmax_t_body,
        grid=(N_PAD // ROWS_PER_BLK,),
        in_specs=[pl.BlockSpec((ROWS_PER_BLK, NUM_CLASSES), lambda i: (i, 0))],
        out_specs=pl.BlockSpec((NUM_CLASSES, ROWS_PER_BLK), lambda i: (0, i)),
        out_shape=jax.ShapeDtypeStruct((NUM_CLASSES, N_PAD), jnp.float32),
    )(class_logits)


def _thresh_body(pt_ref, t_ref):
    p = pt_ref[...]

    def body(k, tu):
        bit = (jnp.uint32(1) << (30 - k).astype(jnp.uint32))
        cand = tu | bit
        cand_f = lax.bitcast_convert_type(cand, jnp.float32)
        cnt = jnp.sum((p >= cand_f).astype(jnp.float32), axis=1, keepdims=True)
        return jnp.where(cnt >= PRE_NMS_TOPK, cand, tu)

    tu = lax.fori_loop(0, 31, body,
                       jnp.zeros((NUM_CLASSES, 1), dtype=jnp.uint32))
    t_ref[...] = lax.bitcast_convert_type(tu, jnp.float32)


def _thresh(probs_t):
    return pl.pallas_call(
        _thresh_body,
        out_shape=jax.ShapeDtypeStruct((NUM_CLASSES, 1), jnp.float32),
    )(probs_t)


def _compact_sc(probs_t, t_pad):
    info = plsc.get_sparse_core_info()
    nc, ns = info.num_cores, info.num_subcores
    nw = nc * ns
    n_chunks = N_PAD // 16

    mesh = plsc.VectorSubcoreMesh(core_axis_name="c", subcore_axis_name="s")

    @functools.partial(
        pl.kernel,
        mesh=mesh,
        out_type=[
            jax.ShapeDtypeStruct((C, SORT_W), jnp.int32),
            jax.ShapeDtypeStruct((C, SORT_W), jnp.float32),
        ],
        scratch_types=[
            pltpu.VMEM((N_PAD,), jnp.float32),
            pltpu.VMEM((SORT_W,), jnp.int32),
            pltpu.VMEM((SORT_W,), jnp.float32),
            pltpu.VMEM((16,), jnp.float32),
        ],
        compiler_params=pltpu.CompilerParams(needs_layout_passes=False,
                                             use_tc_tiling_on_sc=True),
    )
    def k(pt_hbm, t_hbm, oidx_hbm, oval_hbm, col_v, idx_v, val_v, t16_v):
        wid = lax.axis_index("s") * nc + lax.axis_index("c")

        def do_class(cls):
            pltpu.sync_copy(t_hbm.at[cls - 1], t16_v)
            tvec = t16_v[...]

            pad_i = jnp.full((16,), PAD_IDX, dtype=jnp.int32)
            pad_v = jnp.full((16,), NEG, dtype=jnp.float32)

            def initb(i, carry):
                idx_v[pl.ds(pl.multiple_of(i * 16, 16), 16)] = pad_i
                val_v[pl.ds(pl.multiple_of(i * 16, 16), 16)] = pad_v
                return carry

            lax.fori_loop(0, SORT_W // 16, initb, 0)

            pltpu.sync_copy(pt_hbm.at[cls], col_v)

            iota16 = lax.iota(jnp.int32, 16)
            zero = jnp.zeros((16,), dtype=jnp.int32)

            def chunk(i, carry):
                na, nb, idxv = carry
                v = col_v[pl.ds(pl.multiple_of(i * 16, 16), 16)]

                def slow(_):
                    m_a = v > tvec
                    m_b = v == tvec
                    ca = plsc.cumsum(jnp.where(m_a, 1, 0))
                    cb = plsc.cumsum(jnp.where(m_b, 1, 0))
                    pos_a = na + ca - 1
                    pos_b = nb + cb - 1 + REG_B
                    m_b2 = jnp.logical_and(m_b, pos_b < 2 * REG_B)
                    plsc.store_scatter(idx_v, [pos_a], idxv, mask=m_a)
                    plsc.store_scatter(val_v, [pos_a], v, mask=m_a)
                    plsc.store_scatter(idx_v, [pos_b], idxv, mask=m_b2)
                    plsc.store_scatter(val_v, [pos_b], v, mask=m_b2)
                    na2 = na + plsc.all_reduce_population_count(m_a)
                    nb2 = nb + plsc.all_reduce_population_count(m_b)
                    return na2, nb2

                na2, nb2 = lax.cond(jnp.any(v >= tvec), slow,
                                    lambda _: (na, nb), 0)
                return na2, nb2, idxv + 16

            lax.fori_loop(0, n_chunks, chunk, (zero, zero, iota16))

            pltpu.sync_copy(idx_v, oidx_hbm.at[cls - 1])
            pltpu.sync_copy(val_v, oval_hbm.at[cls - 1])

        for t in range(3):
            cls = 1 + wid + nw * t

            @pl.when(cls <= C)
            def _():
                do_class(cls)

    return k(probs_t, t_pad)


def _decode_nms_body(px1, py1, px2, py2, rdx, rdy, rdw, rdh, s_ref,
                     bx1, by1, bx2, by2, out_s,
                     tx1, ty1, tx2, ty2, ts, ot, supp_ref):
    widths = px2[...] - px1[...] + TO_REMOVE
    heights = py2[...] - py1[...] + TO_REMOVE
    ctr_x = px1[...] + 0.5 * widths
    ctr_y = py1[...] + 0.5 * heights
    dx = rdx[...] * (1.0 / WX)
    dy = rdy[...] * (1.0 / WY)
    dw = jnp.minimum(rdw[...] * (1.0 / WW), BBOX_XFORM_CLIP)
    dh = jnp.minimum(rdh[...] * (1.0 / WH), BBOX_XFORM_CLIP)
    pred_ctr_x = dx * widths + ctr_x
    pred_ctr_y = dy * heights + ctr_y
    pred_w = jnp.exp(dw) * widths
    pred_h = jnp.exp(dh) * heights
    x1 = jnp.clip(pred_ctr_x - 0.5 * pred_w, 0.0, IMG_W - TO_REMOVE)
    y1 = jnp.clip(pred_ctr_y - 0.5 * pred_h, 0.0, IMG_H - TO_REMOVE)
    x2 = jnp.clip(pred_ctr_x + 0.5 * pred_w - 1.0, 0.0, IMG_W - TO_REMOVE)
    y2 = jnp.clip(pred_ctr_y + 0.5 * pred_h - 1.0, 0.0, IMG_H - TO_REMOVE)
    bx1[...] = x1
    by1[...] = y1
    bx2[...] = x2
    by2[...] = y2

    x1t = x1.T
    y1t = y1.T
    x2t = x2.T
    y2t = y2.T
    st = s_ref[...].T
    tx1[...] = x1t
    ty1[...] = y1t
    tx2[...] = x2t
    ty2[...] = y2t
    ts[...] = st

    areas = (x2t - x1t + TO_REMOVE) * (y2t - y1t + TO_REMOVE)
    supp_ref[...] = (st <= SCORE_THRESH).astype(jnp.int32)

    def body(i, _):
        x1i = tx1[pl.ds(i, 1), :]
        y1i = ty1[pl.ds(i, 1), :]
        x2i = tx2[pl.ds(i, 1), :]
        y2i = ty2[pl.ds(i, 1), :]
        si = ts[pl.ds(i, 1), :]
        ai = (x2i - x1i + TO_REMOVE) * (y2i - y1i + TO_REMOVE)
        active = supp_ref[pl.ds(i, 1), :] == 0
        w = jnp.maximum(jnp.minimum(x2i, x2t) - jnp.maximum(x1i, x1t) + TO_REMOVE, 0.0)
        h = jnp.maximum(jnp.minimum(y2i, y2t) - jnp.maximum(y1i, y1t) + TO_REMOVE, 0.0)
        inter = w * h
        union = ai + areas - inter
        over = inter > NMS_THRESH * union
        ot[pl.ds(i, 1), :] = jnp.where(active, si, -1e9)
        newsupp = jnp.logical_and(active, over).astype(jnp.int32)
        supp_ref[...] = jnp.maximum(supp_ref[...], newsupp)
        return 0

    lax.fori_loop(0, PRE_NMS_TOPK, body, 0)
    out_s[...] = ot[...].T


def _decode_nms_pallas(planes):
    plane = jax.ShapeDtypeStruct((C, PRE_NMS_TOPK), jnp.float32)
    tplane = pltpu.VMEM((PRE_NMS_TOPK, C), jnp.float32)
    return pl.pallas_call(
        _decode_nms_body,
        out_shape=(plane, plane, plane, plane, plane),
        scratch_shapes=[tplane, tplane, tplane, tplane, tplane, tplane,
                        pltpu.VMEM((PRE_NMS_TOPK, C), jnp.int32)],
    )(*planes)


def kernel(class_logits, box_regression, proposals):
    probs_t = _softmax_t(class_logits)
    t_col = _thresh(probs_t)
    t_rep = jnp.tile(t_col[1:, :], (1, 16))
    cidx, cval = _compact_sc(probs_t, t_rep)
    top_s, pos = lax.top_k(cval, PRE_NMS_TOPK)
    top_i = jnp.take_along_axis(cidx, pos, axis=1)

    cls = jnp.arange(1, NUM_CLASSES, dtype=jnp.int32)[:, None]
    prop_planes = tuple(proposals[:, k][top_i] for k in range(4))
    reg_flat = box_regression.reshape(-1)
    reg_base = top_i * (NUM_CLASSES * 4) + cls * 4
    reg_planes = tuple(reg_flat[reg_base + k] for k in range(4))

    planes = prop_planes + reg_planes + (top_s,)
    bx1, by1, bx2, by2, out_s = _decode_nms_pallas(planes)

    flat_s = out_s.reshape(-1)
    fin_s, fin_i = lax.top_k(flat_s, DETECTIONS_PER_IMG)
    final_boxes = jnp.stack(
        [p.reshape(-1)[fin_i] for p in (bx1, by1, bx2, by2)], axis=1)
    final_labels = jnp.broadcast_to(cls, (C, PRE_NMS_TOPK)).reshape(-1)[fin_i]
    return final_boxes, fin_s, final_labels


# speedup vs baseline: 1.0001x; 1.0001x over previous
import functools
import math

import jax
import jax.numpy as jnp
from jax import lax
from jax.experimental import pallas as pl
from jax.experimental.pallas import tpu as pltpu
from jax.experimental.pallas import tpu_sc as plsc

N = 20000
NUM_CLASSES = 81
IMG_W = 1024.0
IMG_H = 1024.0
SCORE_THRESH = 0.05
NMS_THRESH = 0.5
DETECTIONS_PER_IMG = 100
PRE_NMS_TOPK = 300
BBOX_XFORM_CLIP = math.log(1000.0 / 16.0)
WX, WY, WW, WH = 10.0, 10.0, 5.0, 5.0
TO_REMOVE = 1.0
C = NUM_CLASSES - 1

N_PAD = 20480
ROWS_PER_BLK = 2048
SORT_W = 1024
REG_B = 304
PAD_IDX = 1 << 29
NEG = -3.0e38


def _softmax_t_body(logits_ref, out_ref):
    i = pl.program_id(0)
    x = logits_ref[...]
    m = jnp.max(x, axis=-1, keepdims=True)
    e = jnp.exp(x - m)
    p = e / jnp.sum(e, axis=-1, keepdims=True)
    row = i * ROWS_PER_BLK + lax.broadcasted_iota(jnp.int32, x.shape, 0)
    p = jnp.where(row < N, p, -1.0)
    out_ref[...] = p.T


def _softmax_t(class_logits):
    return pl.pallas_call(
        _softmax_t_body,
        grid=(N_PAD // ROWS_PER_BLK,),
        in_specs=[pl.BlockSpec((ROWS_PER_BLK, NUM_CLASSES), lambda i: (i, 0))],
        out_specs=pl.BlockSpec((NUM_CLASSES, ROWS_PER_BLK), lambda i: (0, i)),
        out_shape=jax.ShapeDtypeStruct((NUM_CLASSES, N_PAD), jnp.float32),
    )(class_logits)


def _thresh_body(pt_ref, t_ref):
    p = pt_ref[...]

    def body(k, tu):
        bit = (jnp.uint32(1) << (30 - k).astype(jnp.uint32))
        cand = tu | bit
        cand_f = lax.bitcast_convert_type(cand, jnp.float32)
        cnt = jnp.sum((p >= cand_f).astype(jnp.float32), axis=1, keepdims=True)
        return jnp.where(cnt >= PRE_NMS_TOPK, cand, tu)

    tu = lax.fori_loop(0, 31, body,
                       jnp.zeros((NUM_CLASSES, 1), dtype=jnp.uint32))
    t_ref[...] = lax.bitcast_convert_type(tu, jnp.float32)


def _thresh(probs_t):
    return pl.pallas_call(
        _thresh_body,
        out_shape=jax.ShapeDtypeStruct((NUM_CLASSES, 1), jnp.float32),
    )(probs_t)


def _compact_sc(probs_t, t_pad):
    info = plsc.get_sparse_core_info()
    nc, ns = info.num_cores, info.num_subcores
    nw = nc * ns
    n_chunks = N_PAD // 16

    mesh = plsc.VectorSubcoreMesh(core_axis_name="c", subcore_axis_name="s")

    @functools.partial(
        pl.kernel,
        mesh=mesh,
        out_type=[
            jax.ShapeDtypeStruct((C, SORT_W), jnp.int32),
            jax.ShapeDtypeStruct((C, SORT_W), jnp.float32),
        ],
        scratch_types=[
            pltpu.VMEM((N_PAD,), jnp.float32),
            pltpu.VMEM((SORT_W,), jnp.int32),
            pltpu.VMEM((SORT_W,), jnp.float32),
            pltpu.VMEM((16,), jnp.float32),
        ],
        compiler_params=pltpu.CompilerParams(needs_layout_passes=False),
    )
    def k(pt_hbm, t_hbm, oidx_hbm, oval_hbm, col_v, idx_v, val_v, t16_v):
        wid = lax.axis_index("s") * nc + lax.axis_index("c")

        def do_class(cls):
            pltpu.sync_copy(t_hbm.at[cls - 1], t16_v)
            tvec = t16_v[...]

            pad_i = jnp.full((16,), PAD_IDX, dtype=jnp.int32)
            pad_v = jnp.full((16,), NEG, dtype=jnp.float32)

            def initb(i, carry):
                idx_v[pl.ds(pl.multiple_of(i * 16, 16), 16)] = pad_i
                val_v[pl.ds(pl.multiple_of(i * 16, 16), 16)] = pad_v
                return carry

            lax.fori_loop(0, SORT_W // 16, initb, 0)

            pltpu.sync_copy(pt_hbm.at[cls], col_v)

            iota16 = lax.iota(jnp.int32, 16)
            zero = jnp.zeros((16,), dtype=jnp.int32)

            def chunk(i, carry):
                na, nb, idxv = carry
                v = col_v[pl.ds(pl.multiple_of(i * 16, 16), 16)]

                def slow(_):
                    m_a = v > tvec
                    m_b = v == tvec
                    ca = plsc.cumsum(jnp.where(m_a, 1, 0))
                    cb = plsc.cumsum(jnp.where(m_b, 1, 0))
                    pos_a = na + ca - 1
                    pos_b = nb + cb - 1 + REG_B
                    m_b2 = jnp.logical_and(m_b, pos_b < 2 * REG_B)
                    plsc.store_scatter(idx_v, [pos_a], idxv, mask=m_a)
                    plsc.store_scatter(val_v, [pos_a], v, mask=m_a)
                    plsc.store_scatter(idx_v, [pos_b], idxv, mask=m_b2)
                    plsc.store_scatter(val_v, [pos_b], v, mask=m_b2)
                    na2 = na + plsc.all_reduce_population_count(m_a)
                    nb2 = nb + plsc.all_reduce_population_count(m_b)
                    return na2, nb2

                na2, nb2 = lax.cond(jnp.any(v >= tvec), slow,
                                    lambda _: (na, nb), 0)
                return na2, nb2, idxv + 16

            lax.fori_loop(0, n_chunks, chunk, (zero, zero, iota16))

            pltpu.sync_copy(idx_v, oidx_hbm.at[cls - 1])
            pltpu.sync_copy(val_v, oval_hbm.at[cls - 1])

        for t in range(3):
            cls = 1 + wid + nw * t

            @pl.when(cls <= C)
            def _():
                do_class(cls)

    return k(probs_t, t_pad)


def _decode_nms_body(px1, py1, px2, py2, rdx, rdy, rdw, rdh, s_ref,
                     bx1, by1, bx2, by2, out_s,
                     tx1, ty1, tx2, ty2, ts, ot, supp_ref):
    widths = px2[...] - px1[...] + TO_REMOVE
    heights = py2[...] - py1[...] + TO_REMOVE
    ctr_x = px1[...] + 0.5 * widths
    ctr_y = py1[...] + 0.5 * heights
    dx = rdx[...] * (1.0 / WX)
    dy = rdy[...] * (1.0 / WY)
    dw = jnp.minimum(rdw[...] * (1.0 / WW), BBOX_XFORM_CLIP)
    dh = jnp.minimum(rdh[...] * (1.0 / WH), BBOX_XFORM_CLIP)
    pred_ctr_x = dx * widths + ctr_x
    pred_ctr_y = dy * heights + ctr_y
    pred_w = jnp.exp(dw) * widths
    pred_h = jnp.exp(dh) * heights
    x1 = jnp.clip(pred_ctr_x - 0.5 * pred_w, 0.0, IMG_W - TO_REMOVE)
    y1 = jnp.clip(pred_ctr_y - 0.5 * pred_h, 0.0, IMG_H - TO_REMOVE)
    x2 = jnp.clip(pred_ctr_x + 0.5 * pred_w - 1.0, 0.0, IMG_W - TO_REMOVE)
    y2 = jnp.clip(pred_ctr_y + 0.5 * pred_h - 1.0, 0.0, IMG_H - TO_REMOVE)
    bx1[...] = x1
    by1[...] = y1
    bx2[...] = x2
    by2[...] = y2

    x1t = x1.T
    y1t = y1.T
    x2t = x2.T
    y2t = y2.T
    st = s_ref[...].T
    tx1[...] = x1t
    ty1[...] = y1t
    tx2[...] = x2t
    ty2[...] = y2t
    ts[...] = st

    areas = (x2t - x1t + TO_REMOVE) * (y2t - y1t + TO_REMOVE)
    supp_ref[...] = (st <= SCORE_THRESH).astype(jnp.int32)

    def body(i, _):
        x1i = tx1[pl.ds(i, 1), :]
        y1i = ty1[pl.ds(i, 1), :]
        x2i = tx2[pl.ds(i, 1), :]
        y2i = ty2[pl.ds(i, 1), :]
        si = ts[pl.ds(i, 1), :]
        ai = (x2i - x1i + TO_REMOVE) * (y2i - y1i + TO_REMOVE)
        active = supp_ref[pl.ds(i, 1), :] == 0
        w = jnp.maximum(jnp.minimum(x2i, x2t) - jnp.maximum(x1i, x1t) + TO_REMOVE, 0.0)
        h = jnp.maximum(jnp.minimum(y2i, y2t) - jnp.maximum(y1i, y1t) + TO_REMOVE, 0.0)
        inter = w * h
        union = ai + areas - inter
        over = inter > NMS_THRESH * union
        ot[pl.ds(i, 1), :] = jnp.where(active, si, -1e9)
        newsupp = jnp.logical_and(active, over).astype(jnp.int32)
        supp_ref[...] = jnp.maximum(supp_ref[...], newsupp)
        return 0

    lax.fori_loop(0, PRE_NMS_TOPK, body, 0)
    out_s[...] = ot[...].T


def _decode_nms_pallas(planes):
    plane = jax.ShapeDtypeStruct((C, PRE_NMS_TOPK), jnp.float32)
    tplane = pltpu.VMEM((PRE_NMS_TOPK, C), jnp.float32)
    return pl.pallas_call(
        _decode_nms_body,
        out_shape=(plane, plane, plane, plane, plane),
        scratch_shapes=[tplane, tplane, tplane, tplane, tplane, tplane,
                        pltpu.VMEM((PRE_NMS_TOPK, C), jnp.int32)],
    )(*planes)


def kernel(class_logits, box_regression, proposals):
    probs_t = _softmax_t(class_logits)
    t_col = _thresh(probs_t)
    t_rep = jnp.tile(t_col[1:, :], (1, 16))
    cidx, cval = _compact_sc(probs_t, t_rep)
    top_s, pos = lax.top_k(cval, PRE_NMS_TOPK)
    top_i = jnp.take_along_axis(cidx, pos, axis=1)

    cls = jnp.arange(1, NUM_CLASSES, dtype=jnp.int32)[:, None]
    prop_planes = tuple(proposals[:, k][top_i] for k in range(4))
    reg_flat = box_regression.reshape(-1)
    reg_base = top_i * (NUM_CLASSES * 4) + cls * 4
    reg_planes = tuple(reg_flat[reg_base + k] for k in range(4))

    planes = prop_planes + reg_planes + (top_s,)
    bx1, by1, bx2, by2, out_s = _decode_nms_pallas(planes)

    flat_s = out_s.reshape(-1)
    fin_s, fin_i = lax.top_k(flat_s, DETECTIONS_PER_IMG)
    final_boxes = jnp.stack(
        [p.reshape(-1)[fin_i] for p in (bx1, by1, bx2, by2)], axis=1)
    final_labels = jnp.broadcast_to(cls, (C, PRE_NMS_TOPK)).reshape(-1)[fin_i]
    return final_boxes, fin_s, final_labels


# speedup vs baseline: 1.1594x; 1.1593x over previous
import functools
import math

import jax
import jax.numpy as jnp
from jax import lax
from jax.experimental import pallas as pl
from jax.experimental.pallas import tpu as pltpu
from jax.experimental.pallas import tpu_sc as plsc

N = 20000
NUM_CLASSES = 81
IMG_W = 1024.0
IMG_H = 1024.0
SCORE_THRESH = 0.05
NMS_THRESH = 0.5
DETECTIONS_PER_IMG = 100
PRE_NMS_TOPK = 300
BBOX_XFORM_CLIP = math.log(1000.0 / 16.0)
WX, WY, WW, WH = 10.0, 10.0, 5.0, 5.0
TO_REMOVE = 1.0
C = NUM_CLASSES - 1

N_PAD = 20480
ROWS_PER_BLK = 2048
SORT_W = 1024
REG_B = 304
PAD_IDX = 1 << 29
NEG = -3.0e38


def _softmax_t_body(logits_ref, out_ref):
    i = pl.program_id(0)
    x = logits_ref[...]
    m = jnp.max(x, axis=-1, keepdims=True)
    e = jnp.exp(x - m)
    p = e / jnp.sum(e, axis=-1, keepdims=True)
    row = i * ROWS_PER_BLK + lax.broadcasted_iota(jnp.int32, x.shape, 0)
    p = jnp.where(row < N, p, -1.0)
    out_ref[...] = p.T


def _softmax_t(class_logits):
    return pl.pallas_call(
        _softmax_t_body,
        grid=(N_PAD // ROWS_PER_BLK,),
        in_specs=[pl.BlockSpec((ROWS_PER_BLK, NUM_CLASSES), lambda i: (i, 0))],
        out_specs=pl.BlockSpec((NUM_CLASSES, ROWS_PER_BLK), lambda i: (0, i)),
        out_shape=jax.ShapeDtypeStruct((NUM_CLASSES, N_PAD), jnp.float32),
    )(class_logits)


def _thresh_body(pt_ref, t_ref):
    p = pt_ref[...]

    def body(k, tu):
        bit = (jnp.uint32(1) << (30 - k).astype(jnp.uint32))
        cand = tu | bit
        cand_f = lax.bitcast_convert_type(cand, jnp.float32)
        cnt = jnp.sum((p >= cand_f).astype(jnp.float32), axis=1, keepdims=True)
        return jnp.where(cnt >= PRE_NMS_TOPK, cand, tu)

    tu = lax.fori_loop(0, 31, body,
                       jnp.zeros((NUM_CLASSES, 1), dtype=jnp.uint32))
    t_ref[...] = lax.bitcast_convert_type(tu, jnp.float32)


def _thresh(probs_t):
    return pl.pallas_call(
        _thresh_body,
        out_shape=jax.ShapeDtypeStruct((NUM_CLASSES, 1), jnp.float32),
    )(probs_t)


def _compact_sc(probs_t, t_pad):
    info = plsc.get_sparse_core_info()
    nc, ns = info.num_cores, info.num_subcores
    nw = nc * ns
    n_chunks = N_PAD // 16

    mesh = plsc.VectorSubcoreMesh(core_axis_name="c", subcore_axis_name="s")

    @functools.partial(
        pl.kernel,
        mesh=mesh,
        out_type=[
            jax.ShapeDtypeStruct((C, SORT_W), jnp.int32),
            jax.ShapeDtypeStruct((C, SORT_W), jnp.float32),
        ],
        scratch_types=[
            pltpu.VMEM((N_PAD,), jnp.float32),
            pltpu.VMEM((SORT_W,), jnp.int32),
            pltpu.VMEM((SORT_W,), jnp.float32),
            pltpu.VMEM((16,), jnp.float32),
        ],
        compiler_params=pltpu.CompilerParams(needs_layout_passes=False),
    )
    def k(pt_hbm, t_hbm, oidx_hbm, oval_hbm, col_v, idx_v, val_v, t16_v):
        wid = lax.axis_index("s") * nc + lax.axis_index("c")

        def do_class(cls):
            pltpu.sync_copy(t_hbm.at[cls - 1], t16_v)
            tvec = t16_v[...]

            pad_i = jnp.full((16,), PAD_IDX, dtype=jnp.int32)
            pad_v = jnp.full((16,), NEG, dtype=jnp.float32)

            def initb(i, carry):
                idx_v[pl.ds(pl.multiple_of(i * 16, 16), 16)] = pad_i
                val_v[pl.ds(pl.multiple_of(i * 16, 16), 16)] = pad_v
                return carry

            lax.fori_loop(0, SORT_W // 16, initb, 0)

            pltpu.sync_copy(pt_hbm.at[cls], col_v)

            iota16 = lax.iota(jnp.int32, 16)
            zero = jnp.zeros((16,), dtype=jnp.int32)

            def chunk(i, carry):
                na, nb, idxv = carry
                v = col_v[pl.ds(pl.multiple_of(i * 16, 16), 16)]
                m_a = v > tvec
                m_b = v == tvec
                ca = plsc.cumsum(jnp.where(m_a, 1, 0))
                cb = plsc.cumsum(jnp.where(m_b, 1, 0))
                pos_a = na + ca - 1
                pos_b = nb + cb - 1 + REG_B
                m_b2 = jnp.logical_and(m_b, pos_b < 2 * REG_B)
                plsc.store_scatter(idx_v, [pos_a], idxv, mask=m_a)
                plsc.store_scatter(val_v, [pos_a], v, mask=m_a)
                plsc.store_scatter(idx_v, [pos_b], idxv, mask=m_b2)
                plsc.store_scatter(val_v, [pos_b], v, mask=m_b2)
                na2 = na + plsc.all_reduce_population_count(m_a)
                nb2 = nb + plsc.all_reduce_population_count(m_b)
                return na2, nb2, idxv + 16

            lax.fori_loop(0, n_chunks, chunk, (zero, zero, iota16))

            pltpu.sync_copy(idx_v, oidx_hbm.at[cls - 1])
            pltpu.sync_copy(val_v, oval_hbm.at[cls - 1])

        for t in range(3):
            cls = 1 + wid + nw * t

            @pl.when(cls <= C)
            def _():
                do_class(cls)

    return k(probs_t, t_pad)


def _decode_nms_body(px1, py1, px2, py2, rdx, rdy, rdw, rdh, s_ref,
                     bx1, by1, bx2, by2, out_s,
                     tx1, ty1, tx2, ty2, ts, ot, supp_ref):
    widths = px2[...] - px1[...] + TO_REMOVE
    heights = py2[...] - py1[...] + TO_REMOVE
    ctr_x = px1[...] + 0.5 * widths
    ctr_y = py1[...] + 0.5 * heights
    dx = rdx[...] * (1.0 / WX)
    dy = rdy[...] * (1.0 / WY)
    dw = jnp.minimum(rdw[...] * (1.0 / WW), BBOX_XFORM_CLIP)
    dh = jnp.minimum(rdh[...] * (1.0 / WH), BBOX_XFORM_CLIP)
    pred_ctr_x = dx * widths + ctr_x
    pred_ctr_y = dy * heights + ctr_y
    pred_w = jnp.exp(dw) * widths
    pred_h = jnp.exp(dh) * heights
    x1 = jnp.clip(pred_ctr_x - 0.5 * pred_w, 0.0, IMG_W - TO_REMOVE)
    y1 = jnp.clip(pred_ctr_y - 0.5 * pred_h, 0.0, IMG_H - TO_REMOVE)
    x2 = jnp.clip(pred_ctr_x + 0.5 * pred_w - 1.0, 0.0, IMG_W - TO_REMOVE)
    y2 = jnp.clip(pred_ctr_y + 0.5 * pred_h - 1.0, 0.0, IMG_H - TO_REMOVE)
    bx1[...] = x1
    by1[...] = y1
    bx2[...] = x2
    by2[...] = y2

    x1t = x1.T
    y1t = y1.T
    x2t = x2.T
    y2t = y2.T
    st = s_ref[...].T
    tx1[...] = x1t
    ty1[...] = y1t
    tx2[...] = x2t
    ty2[...] = y2t
    ts[...] = st

    areas = (x2t - x1t + TO_REMOVE) * (y2t - y1t + TO_REMOVE)
    supp_ref[...] = (st <= SCORE_THRESH).astype(jnp.int32)

    def body(i, _):
        x1i = tx1[pl.ds(i, 1), :]
        y1i = ty1[pl.ds(i, 1), :]
        x2i = tx2[pl.ds(i, 1), :]
        y2i = ty2[pl.ds(i, 1), :]
        si = ts[pl.ds(i, 1), :]
        ai = (x2i - x1i + TO_REMOVE) * (y2i - y1i + TO_REMOVE)
        active = supp_ref[pl.ds(i, 1), :] == 0
        w = jnp.maximum(jnp.minimum(x2i, x2t) - jnp.maximum(x1i, x1t) + TO_REMOVE, 0.0)
        h = jnp.maximum(jnp.minimum(y2i, y2t) - jnp.maximum(y1i, y1t) + TO_REMOVE, 0.0)
        inter = w * h
        union = ai + areas - inter
        over = inter > NMS_THRESH * union
        ot[pl.ds(i, 1), :] = jnp.where(active, si, -1e9)
        newsupp = jnp.logical_and(active, over).astype(jnp.int32)
        supp_ref[...] = jnp.maximum(supp_ref[...], newsupp)
        return 0

    lax.fori_loop(0, PRE_NMS_TOPK, body, 0)
    out_s[...] = ot[...].T


def _decode_nms_pallas(planes):
    plane = jax.ShapeDtypeStruct((C, PRE_NMS_TOPK), jnp.float32)
    tplane = pltpu.VMEM((PRE_NMS_TOPK, C), jnp.float32)
    return pl.pallas_call(
        _decode_nms_body,
        out_shape=(plane, plane, plane, plane, plane),
        scratch_shapes=[tplane, tplane, tplane, tplane, tplane, tplane,
                        pltpu.VMEM((PRE_NMS_TOPK, C), jnp.int32)],
    )(*planes)


def kernel(class_logits, box_regression, proposals):
    probs_t = _softmax_t(class_logits)
    t_col = _thresh(probs_t)
    t_rep = jnp.tile(t_col[1:, :], (1, 16))
    cidx, cval = _compact_sc(probs_t, t_rep)
    top_s, pos = lax.top_k(cval, PRE_NMS_TOPK)
    top_i = jnp.take_along_axis(cidx, pos, axis=1)

    cls = jnp.arange(1, NUM_CLASSES, dtype=jnp.int32)[:, None]
    prop_planes = tuple(proposals[:, k][top_i] for k in range(4))
    reg_flat = box_regression.reshape(-1)
    reg_base = top_i * (NUM_CLASSES * 4) + cls * 4
    reg_planes = tuple(reg_flat[reg_base + k] for k in range(4))

    planes = prop_planes + reg_planes + (top_s,)
    bx1, by1, bx2, by2, out_s = _decode_nms_pallas(planes)

    flat_s = out_s.reshape(-1)
    fin_s, fin_i = lax.top_k(flat_s, DETECTIONS_PER_IMG)
    final_boxes = jnp.stack(
        [p.reshape(-1)[fin_i] for p in (bx1, by1, bx2, by2)], axis=1)
    final_labels = jnp.broadcast_to(cls, (C, PRE_NMS_TOPK)).reshape(-1)[fin_i]
    return final_boxes, fin_s, final_labels
